# baseline (speedup 1.0000x reference)
import functools

import jax
import jax.numpy as jnp
from jax import lax
from jax.experimental import pallas as pl
from jax.experimental.pallas import tpu as pltpu
from jax.experimental.pallas import tpu_sc as plsc

BATCH = 16384
D = 64
L = 16
NW = 32
BPW = BATCH // NW
CH = 128
NCH = BPW // CH

_mesh = plsc.VectorSubcoreMesh(core_axis_name="c", subcore_axis_name="s")


@functools.partial(
    pl.kernel,
    mesh=_mesh,
    out_type=(
        jax.ShapeDtypeStruct((BATCH,), jnp.float32),
        jax.ShapeDtypeStruct((BATCH,), jnp.float32),
    ),
    scratch_types=[
        pltpu.VMEM((NCH, CH), jnp.int32),
        pltpu.VMEM((NCH, CH), jnp.int32),
        pltpu.VMEM((NCH, CH), jnp.int32),
        pltpu.VMEM((BPW, D), jnp.float32),
        pltpu.VMEM((BPW, D), jnp.float32),
        pltpu.VMEM((BPW, D), jnp.float32),
        pltpu.VMEM((BPW,), jnp.float32),
        pltpu.VMEM((BPW,), jnp.float32),
        pltpu.SemaphoreType.DMA,
    ],
    compiler_params=pltpu.CompilerParams(
        use_tc_tiling_on_sc=False, needs_layout_passes=False
    ),
)
def _bprmf_sc(user_hbm, itemi_hbm, itemj_hbm, ut_hbm, it_hbm,
              out_i, out_j,
              uix, iix, jix, urows, irows, jrows, oi, oj, sem):
    wid = lax.axis_index("s") * 2 + lax.axis_index("c")
    base = wid * BPW

    pltpu.sync_copy(user_hbm.at[wid], uix)
    pltpu.sync_copy(itemi_hbm.at[wid], iix)
    pltpu.sync_copy(itemj_hbm.at[wid], jix)

    copies = []
    for k in range(NCH):
        dst = pl.ds(k * CH, CH)
        copies.append(pltpu.async_copy(ut_hbm.at[uix.at[k]], urows.at[dst], sem))
        copies.append(pltpu.async_copy(it_hbm.at[iix.at[k]], irows.at[dst], sem))
        copies.append(pltpu.async_copy(it_hbm.at[jix.at[k]], jrows.at[dst], sem))
    for cp in copies:
        cp.wait()

    iota = jnp.arange(L, dtype=jnp.int32)

    def body(g, carry):
        rowids = g * L + iota
        acc_i = jnp.zeros((L,), jnp.float32)
        acc_j = jnp.zeros((L,), jnp.float32)
        for d in range(D):
            colids = jnp.full((L,), d, dtype=jnp.int32)
            u = plsc.load_gather(urows, [rowids, colids])
            acc_i = acc_i + u * plsc.load_gather(irows, [rowids, colids])
            acc_j = acc_j + u * plsc.load_gather(jrows, [rowids, colids])
        out_off = pl.multiple_of(g * L, L)
        oi[pl.ds(out_off, L)] = acc_i
        oj[pl.ds(out_off, L)] = acc_j
        return carry

    lax.fori_loop(0, BPW // L, body, 0)

    pltpu.sync_copy(oi, out_i.at[pl.ds(base, BPW)])
    pltpu.sync_copy(oj, out_j.at[pl.ds(base, BPW)])


def kernel(user, item_i, item_j, user_table, item_table):
    user_r = user.astype(jnp.int32).reshape(NW, NCH, CH)
    itemi_r = item_i.astype(jnp.int32).reshape(NW, NCH, CH)
    itemj_r = item_j.astype(jnp.int32).reshape(NW, NCH, CH)
    return _bprmf_sc(user_r, itemi_r, itemj_r, user_table, item_table)


# speedup vs baseline: 1.5230x; 1.5230x over previous
import functools

import jax
import jax.numpy as jnp
from jax import lax
from jax.experimental import pallas as pl
from jax.experimental.pallas import tpu as pltpu
from jax.experimental.pallas import tpu_sc as plsc

BATCH = 16384
D = 64
L = 16
NW = 32
BPW = BATCH // NW
CH = 128
NCH = BPW // CH
GPC = CH // L

_mesh = plsc.VectorSubcoreMesh(core_axis_name="c", subcore_axis_name="s")


@functools.partial(
    pl.kernel,
    mesh=_mesh,
    out_type=(
        jax.ShapeDtypeStruct((BATCH,), jnp.float32),
        jax.ShapeDtypeStruct((BATCH,), jnp.float32),
    ),
    scratch_types=[
        pltpu.VMEM((BPW,), jnp.int32),
        pltpu.VMEM((BPW,), jnp.int32),
        pltpu.VMEM((BPW,), jnp.int32),
        pltpu.VMEM((2, CH, D), jnp.float32),
        pltpu.VMEM((2, CH, D), jnp.float32),
        pltpu.VMEM((2, CH, D), jnp.float32),
        pltpu.VMEM((BPW,), jnp.float32),
        pltpu.VMEM((BPW,), jnp.float32),
        pltpu.SemaphoreType.DMA,
        pltpu.SemaphoreType.DMA,
    ],
    compiler_params=pltpu.CompilerParams(needs_layout_passes=False),
)
def _bprmf_sc(user_hbm, itemi_hbm, itemj_hbm, ut_hbm, it_hbm,
              out_i, out_j,
              uix, iix, jix, urows, irows, jrows, oi, oj, sem0, sem1):
    wid = lax.axis_index("s") * 2 + lax.axis_index("c")
    base = wid * BPW

    pltpu.sync_copy(user_hbm.at[pl.ds(base, BPW)], uix)
    pltpu.sync_copy(itemi_hbm.at[pl.ds(base, BPW)], iix)
    pltpu.sync_copy(itemj_hbm.at[pl.ds(base, BPW)], jix)

    sems = (sem0, sem1)

    def issue(c, slot):
        sem = sems[slot]

        def issue_g(g, carry):
            off = pl.multiple_of(c * CH + g * L, L)
            uvec = uix[pl.ds(off, L)]
            ivec = iix[pl.ds(off, L)]
            jvec = jix[pl.ds(off, L)]
            for l in range(L):
                row = g * L + l
                pltpu.async_copy(ut_hbm.at[uvec[l]], urows.at[slot, row], sem)
                pltpu.async_copy(it_hbm.at[ivec[l]], irows.at[slot, row], sem)
                pltpu.async_copy(it_hbm.at[jvec[l]], jrows.at[slot, row], sem)
            return carry

        lax.fori_loop(0, GPC, issue_g, 0)

    def drain(slot):
        sem = sems[slot]

        def drain_g(g, carry):
            for _ in range(3 * L):
                pltpu.make_async_copy(
                    ut_hbm.at[0], urows.at[slot, 0], sem
                ).wait()
            return carry

        lax.fori_loop(0, GPC, drain_g, 0)

    iota = jnp.arange(L, dtype=jnp.int32)

    def compute(c, slot):
        def body(g, carry):
            rowids = g * L + iota
            acc_i = jnp.zeros((L,), jnp.float32)
            acc_j = jnp.zeros((L,), jnp.float32)
            for d in range(D):
                colids = jnp.full((L,), d, dtype=jnp.int32)
                u = plsc.load_gather(urows.at[slot], [rowids, colids])
                acc_i = acc_i + u * plsc.load_gather(
                    irows.at[slot], [rowids, colids])
                acc_j = acc_j + u * plsc.load_gather(
                    jrows.at[slot], [rowids, colids])
            off = pl.multiple_of(c * CH + g * L, L)
            oi[pl.ds(off, L)] = acc_i
            oj[pl.ds(off, L)] = acc_j
            return carry

        lax.fori_loop(0, GPC, body, 0)

    issue(0, 0)
    for c in range(NCH):
        if c + 1 < NCH:
            issue(c + 1, (c + 1) % 2)
        drain(c % 2)
        compute(c, c % 2)

    pltpu.sync_copy(oi, out_i.at[pl.ds(base, BPW)])
    pltpu.sync_copy(oj, out_j.at[pl.ds(base, BPW)])


def kernel(user, item_i, item_j, user_table, item_table):
    return _bprmf_sc(user.astype(jnp.int32), item_i.astype(jnp.int32),
                     item_j.astype(jnp.int32), user_table, item_table)
